# baseline (speedup 1.0000x reference)
import functools

import jax
import jax.numpy as jnp
from jax.experimental import pallas as pl
from jax.experimental.pallas import tpu as pltpu

BLOCK_N = 2048
KSPLIT = 432


def _matmul_bias_kernel(x1_ref, x2_ref, w1_ref, w2_ref, b_ref, o_ref):
    o_ref[...] = (
        jnp.dot(w1_ref[...], x1_ref[...], preferred_element_type=jnp.float32)
        + jnp.dot(w2_ref[...], x2_ref[...], preferred_element_type=jnp.float32)
        + b_ref[...]
    )


@jax.jit
def kernel(x, W_last, b_last, W_dom, b_dom):
    xs = jnp.squeeze(x)
    n, k = xs.shape
    m = W_last.shape[0]
    xt = jnp.swapaxes(xs, 0, 1)
    bc = b_last.reshape(m, 1)
    w1 = W_last[:, :KSPLIT]
    w2 = W_last[:, KSPLIT:]
    grid = (n // BLOCK_N,)
    out_t = pl.pallas_call(
        _matmul_bias_kernel,
        grid=grid,
        in_specs=[
            pl.BlockSpec((KSPLIT, BLOCK_N), lambda j: (0, j)),
            pl.BlockSpec((KSPLIT, BLOCK_N), lambda j: (1, j)),
            pl.BlockSpec((m, KSPLIT), lambda j: (0, 0)),
            pl.BlockSpec((m, KSPLIT), lambda j: (0, 0)),
            pl.BlockSpec((m, 1), lambda j: (0, 0)),
        ],
        out_specs=pl.BlockSpec((m, BLOCK_N), lambda j: (0, j)),
        out_shape=jax.ShapeDtypeStruct((m, n), jnp.float32),
    )(xt, xt, w1, w2, bc)
    return jnp.swapaxes(out_t, 0, 1)


# speedup vs baseline: 1.0130x; 1.0130x over previous
import functools

import jax
import jax.numpy as jnp
from jax.experimental import pallas as pl
from jax.experimental.pallas import tpu as pltpu

BLOCK_N = 2048
NBUF = 3


def _outer_kernel(xt_hbm, w_ref, b_ref, o_hbm):
    k, n = xt_hbm.shape
    m = w_ref.shape[0]

    def body(x_blk, o_blk):
        o_blk[...] = (
            jnp.dot(w_ref[...], x_blk[...], preferred_element_type=jnp.float32)
            + b_ref[...]
        )

    pipeline = pltpu.emit_pipeline(
        body,
        grid=(n // BLOCK_N,),
        in_specs=[
            pl.BlockSpec(
                (k, BLOCK_N),
                lambda j: (0, j),
                pipeline_mode=pl.Buffered(buffer_count=NBUF),
            ),
        ],
        out_specs=[
            pl.BlockSpec((m, BLOCK_N), lambda j: (0, j)),
        ],
    )
    pipeline(xt_hbm, o_hbm)


@jax.jit
def kernel(x, W_last, b_last, W_dom, b_dom):
    xs = jnp.squeeze(x)
    n, k = xs.shape
    m = W_last.shape[0]
    xt = jnp.swapaxes(xs, 0, 1)
    bc = b_last.reshape(m, 1)
    out_t = pl.pallas_call(
        _outer_kernel,
        in_specs=[
            pl.BlockSpec(memory_space=pltpu.MemorySpace.HBM),
            pl.BlockSpec((m, k), lambda: (0, 0)),
            pl.BlockSpec((m, 1), lambda: (0, 0)),
        ],
        out_specs=pl.BlockSpec(memory_space=pltpu.MemorySpace.HBM),
        out_shape=jax.ShapeDtypeStruct((m, n), jnp.float32),
    )(xt, W_last, bc)
    return jnp.swapaxes(out_t, 0, 1)


# speedup vs baseline: 1.0818x; 1.0679x over previous
import functools

import jax
import jax.numpy as jnp
from jax.experimental import pallas as pl
from jax.experimental.pallas import tpu as pltpu

BLOCK_N = 2048


def _matmul_bias_kernel(xt_ref, w_ref, b_ref, o_ref):
    o_ref[...] = (
        jnp.dot(w_ref[...], xt_ref[...], preferred_element_type=jnp.float32)
        + b_ref[...]
    )


@jax.jit
def kernel(x, W_last, b_last, W_dom, b_dom):
    xs = jnp.squeeze(x)
    n, k = xs.shape
    m = W_last.shape[0]
    xt = jnp.swapaxes(xs, 0, 1)
    bc = b_last.reshape(m, 1)
    grid = (n // BLOCK_N,)
    out_t = pl.pallas_call(
        _matmul_bias_kernel,
        grid=grid,
        in_specs=[
            pl.BlockSpec((k, BLOCK_N), lambda j: (0, j)),
            pl.BlockSpec((m, k), lambda j: (0, 0)),
            pl.BlockSpec((m, 1), lambda j: (0, 0)),
        ],
        out_specs=pl.BlockSpec((m, BLOCK_N), lambda j: (0, j)),
        out_shape=jax.ShapeDtypeStruct((m, n), jnp.float32),
        compiler_params=pltpu.CompilerParams(
            dimension_semantics=("arbitrary",),
            disable_bounds_checks=True,
        ),
    )(xt, W_last, bc)
    return jnp.swapaxes(out_t, 0, 1)
